# baseline (speedup 1.0000x reference)
import functools

import jax
import jax.numpy as jnp
from jax import lax
from jax.experimental import pallas as pl
from jax.experimental.pallas import tpu as pltpu
from jax.experimental.pallas import tpu_sc as plsc

EMBED_DIM = 32
OPITCH = 136
NUM_CORES = 2
NUM_SUBCORES = 16
NUM_WORKERS = NUM_CORES * NUM_SUBCORES

UNIT = 128
K = 4


def _build_gather(n_batch: int, n_seq: int):
    n_bc = n_batch // UNIT
    total_units = n_seq * n_bc
    assert total_units % (NUM_WORKERS * K) == 0
    units_per_w = total_units // NUM_WORKERS
    num_groups = units_per_w // K

    mesh = plsc.VectorSubcoreMesh(core_axis_name="c", subcore_axis_name="s")

    scratch = (
        [pltpu.VMEM((UNIT,), jnp.int32) for _ in range(K)]
        + [pltpu.VMEM((UNIT, EMBED_DIM), jnp.float32) for _ in range(K)]
        + [pltpu.VMEM((EMBED_DIM, OPITCH), jnp.float32) for _ in range(K)]
        + [pltpu.SemaphoreType.DMA for _ in range(2 * K)]
    )

    @functools.partial(
        pl.kernel,
        mesh=mesh,
        out_type=jax.ShapeDtypeStruct((n_seq, EMBED_DIM, n_batch),
                                      jnp.float32),
        scratch_types=scratch,
        compiler_params=pltpu.CompilerParams(
            use_tc_tiling_on_sc=False, needs_layout_passes=False),
    )
    def gather_kernel(idx_hbm, table_hbm, out_hbm, *bufs):
        idx_v = bufs[:K]
        g_v = bufs[K:2 * K]
        o_v = bufs[2 * K:3 * K]
        gsem = bufs[3 * K:4 * K]
        osem = bufs[4 * K:5 * K]

        wid = lax.axis_index("s") * NUM_CORES + lax.axis_index("c")
        ubase = wid * units_per_w
        lane = lax.iota(jnp.int32, 16)
        rows = [lane + jnp.int32(dg * 16) for dg in range(EMBED_DIM // 16)]

        def unit_coords(u):
            j = u // n_bc
            bc = u - j * n_bc
            return j, bc

        def fire(u, b):
            pltpu.sync_copy(idx_hbm.at[pl.ds(u * UNIT, UNIT)], idx_v[b])
            pltpu.async_copy(table_hbm.at[idx_v[b]], g_v[b], gsem[b])

        def extract_and_store(u, b):
            pltpu.make_async_copy(
                table_hbm.at[idx_v[b]], g_v[b], gsem[b]).wait()
            for bb in range(UNIT):
                col = jnp.full((16,), bb, jnp.int32)
                for dg in range(EMBED_DIM // 16):
                    vals = g_v[b][bb, pl.ds(dg * 16, 16)]
                    plsc.store_scatter(o_v[b], [rows[dg], col], vals)
            j, bc = unit_coords(u)
            pltpu.async_copy(
                o_v[b].at[:, pl.ds(0, UNIT)],
                out_hbm.at[j, :, pl.ds(bc * UNIT, UNIT)], osem[b])

        def drain(u, b):
            j, bc = unit_coords(u)
            pltpu.make_async_copy(
                o_v[b].at[:, pl.ds(0, UNIT)],
                out_hbm.at[j, :, pl.ds(bc * UNIT, UNIT)], osem[b]).wait()

        def group_body(grp, carry):
            u0 = ubase + grp * K
            for b in range(K):
                fire(u0 + b, b)
            for b in range(K):
                extract_and_store(u0 + b, b)
            for b in range(K):
                drain(u0 + b, b)
            return carry

        lax.fori_loop(0, num_groups, group_body, 0)

    return gather_kernel


def kernel(x, table):
    n_batch, n_seq = x.shape
    idx = x.T.reshape(-1).astype(jnp.int32)
    out3 = _build_gather(n_batch, n_seq)(idx, table)
    return out3.transpose(2, 0, 1)


# speedup vs baseline: 1.0374x; 1.0374x over previous
import functools

import jax
import jax.numpy as jnp
from jax import lax
from jax.experimental import pallas as pl
from jax.experimental.pallas import tpu as pltpu
from jax.experimental.pallas import tpu_sc as plsc

EMBED_DIM = 32
OPITCH = 136
NUM_CORES = 2
NUM_SUBCORES = 16
NUM_WORKERS = NUM_CORES * NUM_SUBCORES

UNIT = 256
K = 2


def _build_gather(n_batch: int, n_seq: int):
    n_bc = n_batch // UNIT
    total_units = n_seq * n_bc
    assert total_units % (NUM_WORKERS * K) == 0
    units_per_w = total_units // NUM_WORKERS
    num_groups = units_per_w // K

    mesh = plsc.VectorSubcoreMesh(core_axis_name="c", subcore_axis_name="s")

    scratch = (
        [pltpu.VMEM((UNIT,), jnp.int32) for _ in range(K)]
        + [pltpu.VMEM((UNIT, EMBED_DIM), jnp.float32) for _ in range(K)]
        + [pltpu.VMEM((EMBED_DIM, OPITCH), jnp.float32) for _ in range(K)]
        + [pltpu.SemaphoreType.DMA for _ in range(2 * K)]
    )

    @functools.partial(
        pl.kernel,
        mesh=mesh,
        out_type=jax.ShapeDtypeStruct((n_seq, EMBED_DIM, n_batch),
                                      jnp.float32),
        scratch_types=scratch,
        compiler_params=pltpu.CompilerParams(
            use_tc_tiling_on_sc=False, needs_layout_passes=False),
    )
    def gather_kernel(idx_hbm, table_hbm, out_hbm, *bufs):
        idx_v = bufs[:K]
        g_v = bufs[K:2 * K]
        o_v = bufs[2 * K:3 * K]
        gsem = bufs[3 * K:4 * K]
        osem = bufs[4 * K:5 * K]

        wid = lax.axis_index("s") * NUM_CORES + lax.axis_index("c")
        ubase = wid * units_per_w
        lane = lax.iota(jnp.int32, 16)
        rows = [lane + jnp.int32(dg * 16) for dg in range(EMBED_DIM // 16)]

        def unit_coords(u):
            j = u // n_bc
            bc = u - j * n_bc
            return j, bc

        def fire(u, b):
            pltpu.sync_copy(idx_hbm.at[pl.ds(u * UNIT, UNIT)], idx_v[b])
            pltpu.async_copy(table_hbm.at[idx_v[b]], g_v[b], gsem[b])

        def extract_and_store(u, b):
            pltpu.make_async_copy(
                table_hbm.at[idx_v[b]], g_v[b], gsem[b]).wait()
            for bb in range(UNIT):
                col = jnp.full((16,), bb, jnp.int32)
                for dg in range(EMBED_DIM // 16):
                    vals = g_v[b][bb, pl.ds(dg * 16, 16)]
                    plsc.store_scatter(o_v[b], [rows[dg], col], vals)
            j, bc = unit_coords(u)
            pltpu.async_copy(
                o_v[b].at[:, pl.ds(0, UNIT)],
                out_hbm.at[j, :, pl.ds(bc * UNIT, UNIT)], osem[b])

        def drain(u, b):
            j, bc = unit_coords(u)
            pltpu.make_async_copy(
                o_v[b].at[:, pl.ds(0, UNIT)],
                out_hbm.at[j, :, pl.ds(bc * UNIT, UNIT)], osem[b]).wait()

        def group_body(grp, carry):
            u0 = ubase + grp * K
            for b in range(K):
                fire(u0 + b, b)
            for b in range(K):
                extract_and_store(u0 + b, b)
            for b in range(K):
                drain(u0 + b, b)
            return carry

        lax.fori_loop(0, num_groups, group_body, 0)

    return gather_kernel


def kernel(x, table):
    n_batch, n_seq = x.shape
    idx = x.T.reshape(-1).astype(jnp.int32)
    out3 = _build_gather(n_batch, n_seq)(idx, table)
    return out3.transpose(2, 0, 1)


# speedup vs baseline: 1.0402x; 1.0027x over previous
import functools

import jax
import jax.numpy as jnp
from jax import lax
from jax.experimental import pallas as pl
from jax.experimental.pallas import tpu as pltpu
from jax.experimental.pallas import tpu_sc as plsc

EMBED_DIM = 32
NUM_CORES = 2
NUM_SUBCORES = 16
NUM_WORKERS = NUM_CORES * NUM_SUBCORES

UNIT = 256
OPITCH = UNIT + 8
K = 2


def _build_gather(n_batch: int, n_seq: int):
    n_bc = n_batch // UNIT
    total_units = n_seq * n_bc
    assert total_units % (NUM_WORKERS * K) == 0
    units_per_w = total_units // NUM_WORKERS
    num_groups = units_per_w // K

    mesh = plsc.VectorSubcoreMesh(core_axis_name="c", subcore_axis_name="s")

    scratch = (
        [pltpu.VMEM((UNIT,), jnp.int32) for _ in range(K)]
        + [pltpu.VMEM((UNIT, EMBED_DIM), jnp.float32) for _ in range(K)]
        + [pltpu.VMEM((EMBED_DIM, OPITCH), jnp.float32) for _ in range(K)]
        + [pltpu.SemaphoreType.DMA for _ in range(2 * K)]
    )

    @functools.partial(
        pl.kernel,
        mesh=mesh,
        out_type=jax.ShapeDtypeStruct((n_seq, EMBED_DIM, n_batch),
                                      jnp.float32),
        scratch_types=scratch,
        compiler_params=pltpu.CompilerParams(
            use_tc_tiling_on_sc=False, needs_layout_passes=False),
    )
    def gather_kernel(idx_hbm, table_hbm, out_hbm, *bufs):
        idx_v = bufs[:K]
        g_v = bufs[K:2 * K]
        o_v = bufs[2 * K:3 * K]
        gsem = bufs[3 * K:4 * K]
        osem = bufs[4 * K:5 * K]

        wid = lax.axis_index("s") * NUM_CORES + lax.axis_index("c")
        ubase = wid * units_per_w
        lane = lax.iota(jnp.int32, 16)
        rows = [lane + jnp.int32(dg * 16) for dg in range(EMBED_DIM // 16)]

        def unit_coords(u):
            j = u // n_bc
            bc = u - j * n_bc
            return j, bc

        def fire(u, b):
            pltpu.sync_copy(idx_hbm.at[pl.ds(u * UNIT, UNIT)], idx_v[b])
            pltpu.async_copy(table_hbm.at[idx_v[b]], g_v[b], gsem[b])

        def extract_and_store(u, b):
            pltpu.make_async_copy(
                table_hbm.at[idx_v[b]], g_v[b], gsem[b]).wait()
            for bb in range(UNIT):
                col = jnp.full((16,), bb, jnp.int32)
                for dg in range(EMBED_DIM // 16):
                    vals = g_v[b][bb, pl.ds(dg * 16, 16)]
                    plsc.store_scatter(o_v[b], [rows[dg], col], vals)
            j, bc = unit_coords(u)
            pltpu.async_copy(
                o_v[b].at[:, pl.ds(0, UNIT)],
                out_hbm.at[j, :, pl.ds(bc * UNIT, UNIT)], osem[b])

        def drain(u, b):
            j, bc = unit_coords(u)
            pltpu.make_async_copy(
                o_v[b].at[:, pl.ds(0, UNIT)],
                out_hbm.at[j, :, pl.ds(bc * UNIT, UNIT)], osem[b]).wait()

        def group_body(grp, carry):
            u0 = ubase + grp * K
            for b in range(K):
                fire(u0 + b, b)
            for b in range(K):
                extract_and_store(u0 + b, b)
            for b in range(K):
                drain(u0 + b, b)
            return carry

        lax.fori_loop(0, num_groups, group_body, 0)

    return gather_kernel


def kernel(x, table):
    n_batch, n_seq = x.shape
    idx = x.T.reshape(-1).astype(jnp.int32)
    out3 = _build_gather(n_batch, n_seq)(idx, table)
    return out3.transpose(2, 0, 1)


# speedup vs baseline: 1.0629x; 1.0219x over previous
import functools

import jax
import jax.numpy as jnp
from jax import lax
from jax.experimental import pallas as pl
from jax.experimental.pallas import tpu as pltpu
from jax.experimental.pallas import tpu_sc as plsc

EMBED_DIM = 32
NUM_CORES = 2
NUM_SUBCORES = 16
NUM_WORKERS = NUM_CORES * NUM_SUBCORES

UNIT = 512
OPITCH = UNIT + 8
K = 2


def _build_gather(n_batch: int, n_seq: int):
    n_bc = n_batch // UNIT
    total_units = n_seq * n_bc
    assert total_units % (NUM_WORKERS * K) == 0
    units_per_w = total_units // NUM_WORKERS
    num_groups = units_per_w // K

    mesh = plsc.VectorSubcoreMesh(core_axis_name="c", subcore_axis_name="s")

    scratch = (
        [pltpu.VMEM((UNIT,), jnp.int32) for _ in range(K)]
        + [pltpu.VMEM((UNIT, EMBED_DIM), jnp.float32) for _ in range(K)]
        + [pltpu.VMEM((EMBED_DIM, OPITCH), jnp.float32) for _ in range(K)]
        + [pltpu.SemaphoreType.DMA for _ in range(2 * K)]
    )

    @functools.partial(
        pl.kernel,
        mesh=mesh,
        out_type=jax.ShapeDtypeStruct((n_seq, EMBED_DIM, n_batch),
                                      jnp.float32),
        scratch_types=scratch,
        compiler_params=pltpu.CompilerParams(
            use_tc_tiling_on_sc=False, needs_layout_passes=False),
    )
    def gather_kernel(idx_hbm, table_hbm, out_hbm, *bufs):
        idx_v = bufs[:K]
        g_v = bufs[K:2 * K]
        o_v = bufs[2 * K:3 * K]
        gsem = bufs[3 * K:4 * K]
        osem = bufs[4 * K:5 * K]

        wid = lax.axis_index("s") * NUM_CORES + lax.axis_index("c")
        ubase = wid * units_per_w
        lane = lax.iota(jnp.int32, 16)
        rows = [lane + jnp.int32(dg * 16) for dg in range(EMBED_DIM // 16)]

        def unit_coords(u):
            j = u // n_bc
            bc = u - j * n_bc
            return j, bc

        def fire(u, b):
            pltpu.sync_copy(idx_hbm.at[pl.ds(u * UNIT, UNIT)], idx_v[b])
            pltpu.async_copy(table_hbm.at[idx_v[b]], g_v[b], gsem[b])

        def extract_and_store(u, b):
            pltpu.make_async_copy(
                table_hbm.at[idx_v[b]], g_v[b], gsem[b]).wait()
            for bb in range(UNIT):
                col = jnp.full((16,), bb, jnp.int32)
                for dg in range(EMBED_DIM // 16):
                    vals = g_v[b][bb, pl.ds(dg * 16, 16)]
                    plsc.store_scatter(o_v[b], [rows[dg], col], vals)
            j, bc = unit_coords(u)
            pltpu.async_copy(
                o_v[b].at[:, pl.ds(0, UNIT)],
                out_hbm.at[j, :, pl.ds(bc * UNIT, UNIT)], osem[b])

        def drain(u, b):
            j, bc = unit_coords(u)
            pltpu.make_async_copy(
                o_v[b].at[:, pl.ds(0, UNIT)],
                out_hbm.at[j, :, pl.ds(bc * UNIT, UNIT)], osem[b]).wait()

        def group_body(grp, carry):
            u0 = ubase + grp * K
            for b in range(K):
                fire(u0 + b, b)
            for b in range(K):
                extract_and_store(u0 + b, b)
            for b in range(K):
                drain(u0 + b, b)
            return carry

        lax.fori_loop(0, num_groups, group_body, 0)

    return gather_kernel


def kernel(x, table):
    n_batch, n_seq = x.shape
    idx = x.T.reshape(-1).astype(jnp.int32)
    out3 = _build_gather(n_batch, n_seq)(idx, table)
    return out3.transpose(2, 0, 1)
